# baseline (speedup 1.0000x reference)
import functools

import jax
import jax.numpy as jnp
from jax import lax
from jax.experimental import pallas as pl
from jax.experimental.pallas import tpu as pltpu
from jax.experimental.pallas import tpu_sc as plsc

N = 10000
E = 320000
DE = 16
H = 128
HP = H // 2

NW = 32
CBB = 128
CBD = 128
NCK = E // CBD
BCT = NCK // NW
XTR = NCK - BCT * NW
EH = E // 2
NCKH = NCK // 2
BCTH = NCKH // 16
XTRH = NCKH - BCTH * 16
EBH = 1280
NPAD = 10240
RPT = NPAD // 16
ZB = 128
NB = 2000
EB = 2000


def _ln(h, g, b):
    mu = jnp.mean(h, axis=-1, keepdims=True)
    var = jnp.mean((h - mu) ** 2, axis=-1, keepdims=True)
    return (h - mu) * lax.rsqrt(var + 1e-5) * g + b


def _swish(x):
    return x * jax.nn.sigmoid(x)


def _dot(a, b):
    return jnp.dot(a, b, preferred_element_type=jnp.float32)


def _dotb(a, b):
    return jnp.dot(a.astype(jnp.bfloat16), b.astype(jnp.bfloat16),
                   preferred_element_type=jnp.float32)


def _pack_pair(p):
    ua = lax.bitcast_convert_type(p[:, :HP], jnp.uint32)
    ub = lax.bitcast_convert_type(p[:, HP:], jnp.uint32)
    lo = (ua + jnp.uint32(0x8000)) >> 16
    hi = (ub + jnp.uint32(0x8000)) & jnp.uint32(0xFFFF0000)
    return lax.bitcast_convert_type(lo | hi, jnp.float32)


def _unpack_pair(p_ref):
    u = lax.bitcast_convert_type(p_ref[...], jnp.uint32)
    lo = lax.bitcast_convert_type(u << 16, jnp.float32)
    hi = lax.bitcast_convert_type(u & jnp.uint32(0xFFFF0000), jnp.float32)
    return lo, hi


def _node_embed_body(x_ref, w1_ref, b1_ref, w2_ref, b2_ref, gm_ref, bt_ref,
                     ws_ref, wr_ref, nf_ref, ps_ref, pr_ref):
    h = _dot(x_ref[...], w1_ref[...]) + b1_ref[...]
    h = _swish(h)
    h = _dot(h, w2_ref[...]) + b2_ref[...]
    nf = _ln(h, gm_ref[...], bt_ref[...])
    nf_ref[...] = nf
    ps_ref[...] = _pack_pair(_dot(nf, ws_ref[...]))
    pr_ref[...] = _pack_pair(_dot(nf, wr_ref[...]))


def _unpack_pair(p_ref):
    u = lax.bitcast_convert_type(p_ref[...], jnp.uint32)
    lo = lax.bitcast_convert_type(u << 16, jnp.float32)
    hi = lax.bitcast_convert_type(u & jnp.uint32(0xFFFF0000), jnp.float32)
    return lo, hi


def _edge_body(xl_ref, xh_ref, gs_ref, gr_ref, we1, be1, we2, be2, ge, bte,
               wp1, bp1, wp2, bp2, gp, btp, out_ref):
    x = jnp.concatenate([xl_ref[...], xh_ref[...]], axis=0)
    h = _dot(x, we1[...]) + be1[...]
    h = _swish(h)
    h = _dotb(h, we2[...]) + be2[...]
    ef = _ln(h, ge[...], bte[...])
    slo, shi = _unpack_pair(gs_ref)
    rlo, rhi = _unpack_pair(gr_ref)
    g = jnp.concatenate(
        [jnp.concatenate([slo[:, :HP] + rlo[:, :HP],
                          shi[:, :HP] + rhi[:, :HP]], axis=1),
         jnp.concatenate([slo[:, HP:] + rlo[:, HP:],
                          shi[:, HP:] + rhi[:, HP:]], axis=1)], axis=0)
    z = _dotb(ef, wp1[...]) + g + bp1[...]
    z = _swish(z)
    o = _dotb(z, wp2[...]) + bp2[...]
    out_ref[...] = _ln(o, gp[...], btp[...]).reshape(2, EBH, H)


def _node_update_body(nf_ref, a0_ref, a1_ref, wn1a, wn1b, b1n, wn2, b2n,
                      gn, btn, wo1, bo1, wo2, bo2, out_ref):
    nf = nf_ref[...]
    agg = a0_ref[0] + a1_ref[0]
    h = _dot(nf, wn1a[...]) + _dot(agg, wn1b[...]) + b1n[...]
    h = _swish(h)
    y = _ln(_dot(h, wn2[...]) + b2n[...], gn[...], btn[...])
    r = y + nf
    o = _swish(_dot(r, wo1[...]) + bo1[...])
    out_ref[...] = _dot(o, wo2[...]) + bo2[...]


@functools.cache
def _sc_kernels():
    mesh = plsc.VectorSubcoreMesh(core_axis_name="c", subcore_axis_name="s")

    @functools.partial(
        pl.kernel,
        mesh=mesh,
        out_type=[jax.ShapeDtypeStruct((EH, H), jnp.float32),
                  jax.ShapeDtypeStruct((EH, H), jnp.float32)],
        scratch_types=[
            pltpu.VMEM((BCTH * CBB,), jnp.int32),
            pltpu.VMEM((BCTH * CBB,), jnp.int32),
            pltpu.VMEM((CBB,), jnp.int32),
            pltpu.VMEM((CBB,), jnp.int32),
            pltpu.VMEM((CBB, HP), jnp.float32),
            pltpu.VMEM((CBB, HP), jnp.float32),
            pltpu.VMEM((CBB, HP), jnp.float32),
            pltpu.VMEM((CBB, HP), jnp.float32),
        ] + [pltpu.SemaphoreType.DMA] * 8,
        compiler_params=pltpu.CompilerParams(use_tc_tiling_on_sc=False),
    )
    def _sc_gather(sidx, ridx, tabs, tabr, gs, gr,
                   sidx_v, ridx_v, stail_v, rtail_v, rs0, rs1, rr0, rr1,
                   sgs0, sgs1, sgr0, sgr1, sws0, sws1, swr0, swr1):
        c = lax.axis_index("c")
        s = lax.axis_index("s")
        wid = s * 2 + c
        half = wid // 16
        widh = wid % 16
        starth = BCTH * widh + jnp.minimum(widh, XTRH)
        ebase = half * EH + starth * CBB
        pltpu.sync_copy(sidx.at[pl.ds(ebase, BCTH * CBB)], sidx_v)
        pltpu.sync_copy(ridx.at[pl.ds(ebase, BCTH * CBB)], ridx_v)
        col = half * HP
        rows_s = (rs0, rs1)
        rows_r = (rr0, rr1)
        sem_gs = (sgs0, sgs1)
        sem_gr = (sgr0, sgr1)
        sem_ws = (sws0, sws1)
        sem_wr = (swr0, swr1)

        def issue_g(j, b):
            pltpu.async_copy(tabs.at[sidx_v.at[pl.ds(j * CBB, CBB)]],
                             rows_s[b], sem_gs[b])
            pltpu.async_copy(tabr.at[ridx_v.at[pl.ds(j * CBB, CBB)]],
                             rows_r[b], sem_gr[b])

        def wait_g(b):
            pltpu.make_async_copy(tabs.at[pl.ds(0, CBB)], rows_s[b],
                                  sem_gs[b]).wait()
            pltpu.make_async_copy(tabr.at[pl.ds(0, CBB)], rows_r[b],
                                  sem_gr[b]).wait()

        def issue_w(j, b):
            r0w = (starth + j) * CBB
            pltpu.async_copy(rows_s[b],
                             gs.at[pl.ds(r0w, CBB), pl.ds(col, HP)],
                             sem_ws[b])
            pltpu.async_copy(rows_r[b],
                             gr.at[pl.ds(r0w, CBB), pl.ds(col, HP)],
                             sem_wr[b])

        def wait_w(b):
            pltpu.make_async_copy(rows_s[b],
                                  gs.at[pl.ds(0, CBB), pl.ds(0, HP)],
                                  sem_ws[b]).wait()
            pltpu.make_async_copy(rows_r[b],
                                  gr.at[pl.ds(0, CBB), pl.ds(0, HP)],
                                  sem_wr[b]).wait()

        issue_g(0, 0)

        def body(g, carry):
            j0 = 2 * g

            @pl.when(g >= 1)
            def _():
                wait_w(1)

            issue_g(j0 + 1, 1)
            wait_g(0)
            issue_w(j0, 0)

            @pl.when(g <= BCTH // 2 - 2)
            def _():
                wait_w(0)
                issue_g(j0 + 2, 0)

            wait_g(1)
            issue_w(j0 + 1, 1)
            return carry

        lax.fori_loop(0, BCTH // 2, body, 0)
        wait_w(0)
        wait_w(1)

        @pl.when(widh < XTRH)
        def _():
            jt = starth + BCTH
            et = half * EH + jt * CBB
            pltpu.sync_copy(sidx.at[pl.ds(et, CBB)], stail_v)
            pltpu.sync_copy(ridx.at[pl.ds(et, CBB)], rtail_v)
            pltpu.async_copy(tabs.at[stail_v], rs0, sgs0).wait()
            pltpu.async_copy(tabr.at[rtail_v], rr0, sgr0).wait()
            pltpu.sync_copy(rs0, gs.at[pl.ds(jt * CBB, CBB), pl.ds(col, HP)])
            pltpu.sync_copy(rr0, gr.at[pl.ds(jt * CBB, CBB), pl.ds(col, HP)])

    @functools.partial(
        pl.kernel,
        mesh=mesh,
        out_type=jax.ShapeDtypeStruct((2, NPAD, H), jnp.float32),
        scratch_types=[
            pltpu.VMEM((BCT + 2, CBD), jnp.int32),
            pltpu.VMEM((ZB, H), jnp.float32),
            pltpu.VMEM((ZB, H), jnp.float32),
            pltpu.VMEM_SHARED((NPAD, H), jnp.float32),
            pltpu.SemaphoreType.DMA,
            pltpu.SemaphoreType.DMA,
        ],
    )
    def _sc_segsum(ridx, el, zer, out, ridx_v, r0, r1, agg_sp,
                   sem0, sem1):
        c = lax.axis_index("c")
        s = lax.axis_index("s")
        wid = s * 2 + c
        start = BCT * wid + jnp.minimum(wid, XTR)
        pltpu.sync_copy(ridx.at[wid], ridx_v)
        pltpu.sync_copy(zer, r0)
        for z in range(RPT // ZB):
            pltpu.sync_copy(r0, agg_sp.at[pl.ds(s * RPT + z * ZB, ZB)])
        plsc.subcore_barrier()
        rows = (r0, r1)
        sems = (sem0, sem1)

        def issue_r(j, b):
            pltpu.async_copy(el.at[pl.ds((start + j) * CBD, CBD)],
                             rows[b], sems[b])

        def wait_r(b):
            pltpu.make_async_copy(el.at[pl.ds(0, CBD)], rows[b],
                                  sems[b]).wait()

        issue_r(0, 0)

        def body(g, carry):
            j0 = 2 * g
            issue_r(j0 + 1, 1)
            wait_r(0)
            pltpu.sync_copy(rows[0], agg_sp.at[ridx_v.at[j0]], add=True)

            @pl.when(g <= BCT // 2 - 2)
            def _():
                issue_r(j0 + 2, 0)

            wait_r(1)
            pltpu.sync_copy(rows[1], agg_sp.at[ridx_v.at[j0 + 1]], add=True)
            return carry

        lax.fori_loop(0, BCT // 2, body, 0)

        @pl.when(wid < XTR)
        def _():
            pltpu.sync_copy(el.at[pl.ds((start + BCT) * CBD, CBD)], r0)
            pltpu.sync_copy(r0, agg_sp.at[ridx_v.at[BCT]], add=True)

        plsc.subcore_barrier()
        for z in range(RPT // ZB):
            pltpu.sync_copy(agg_sp.at[pl.ds(s * RPT + z * ZB, ZB)], r0)
            pltpu.sync_copy(r0, out.at[c, pl.ds(s * RPT + z * ZB, ZB)])

    return _sc_gather, _sc_segsum


def _row_spec(block, idx_fn):
    return pl.BlockSpec(block, idx_fn)


def kernel(edge_idx, edge_features, node_features, params):
    pe = params["embed_edge"]
    pn = params["embed_node"]
    pp = params["proc_edge"]
    pq = params["proc_node"]
    po = params["node_out"]

    r1 = lambda v: v.reshape(1, H)
    senders = edge_idx[0]
    receivers = edge_idx[1]

    w1e = pp["W1"][:H]
    w1s = pp["W1"][H:2 * H]
    w1r = pp["W1"][2 * H:]
    wq1a = pq["W1"][:H]
    wq1b = pq["W1"][H:]

    wspec = lambda shape: pl.BlockSpec(shape, lambda i: (0, 0))

    nf, tabs, tabr = pl.pallas_call(
        _node_embed_body,
        grid=(N // NB,),
        in_specs=[
            _row_spec((NB, H), lambda i: (i, 0)),
            wspec((H, H)), wspec((1, H)), wspec((H, H)), wspec((1, H)),
            wspec((1, H)), wspec((1, H)), wspec((H, H)), wspec((H, H)),
        ],
        out_specs=[_row_spec((NB, H), lambda i: (i, 0)),
                   _row_spec((NB, HP), lambda i: (i, 0)),
                   _row_spec((NB, HP), lambda i: (i, 0))],
        out_shape=[jax.ShapeDtypeStruct((N, H), jnp.float32),
                   jax.ShapeDtypeStruct((N, HP), jnp.float32),
                   jax.ShapeDtypeStruct((N, HP), jnp.float32)],
    )(node_features, pn["W1"], r1(pn["b1"]), pn["W2"], r1(pn["b2"]),
      r1(pn["gamma"]), r1(pn["beta"]), w1s, w1r)

    sc_gather, sc_segsum = _sc_kernels()
    gs, gr = sc_gather(senders, receivers, tabs, tabr)

    el3 = pl.pallas_call(
        _edge_body,
        grid=(EH // EBH,),
        in_specs=[
            pl.BlockSpec((EBH, DE), lambda i: (i, 0)),
            pl.BlockSpec((EBH, DE), lambda i: (EH // EBH + i, 0)),
            pl.BlockSpec((EBH, H), lambda i: (i, 0)),
            pl.BlockSpec((EBH, H), lambda i: (i, 0)),
            wspec((DE, H)), wspec((1, H)), wspec((H, H)), wspec((1, H)),
            wspec((1, H)), wspec((1, H)),
            wspec((H, H)), wspec((1, H)), wspec((H, H)), wspec((1, H)),
            wspec((1, H)), wspec((1, H)),
        ],
        out_specs=pl.BlockSpec((2, EBH, H), lambda i: (0, i, 0)),
        out_shape=jax.ShapeDtypeStruct((2, EH, H), jnp.float32),
    )(edge_features, edge_features, gs, gr,
      pe["W1"], r1(pe["b1"]), pe["W2"], r1(pe["b2"]),
      r1(pe["gamma"]), r1(pe["beta"]),
      w1e, r1(pp["b1"]), pp["W2"], r1(pp["b2"]),
      r1(pp["gamma"]), r1(pp["beta"]))
    el = el3.reshape(E, H)

    starts = BCT * jnp.arange(NW) + jnp.minimum(jnp.arange(NW), XTR)
    ridx_d = jnp.pad(receivers.reshape(NCK, CBD), ((0, 8), (0, 0)))[
        starts[:, None] + jnp.arange(BCT + 2)[None, :]]
    zer = jnp.zeros((ZB, H), jnp.float32)
    parts = sc_segsum(ridx_d, el, zer)

    nl = pl.pallas_call(
        _node_update_body,
        grid=(N // NB,),
        in_specs=[
            _row_spec((NB, H), lambda i: (i, 0)),
            pl.BlockSpec((1, NB, H), lambda i: (0, i, 0)),
            pl.BlockSpec((1, NB, H), lambda i: (1, i, 0)),
            wspec((H, H)), wspec((H, H)), wspec((1, H)),
            wspec((H, H)), wspec((1, H)), wspec((1, H)), wspec((1, H)),
            wspec((H, H)), wspec((1, H)), wspec((H, H)), wspec((1, H)),
        ],
        out_specs=_row_spec((NB, H), lambda i: (i, 0)),
        out_shape=jax.ShapeDtypeStruct((N, H), jnp.float32),
    )(nf, parts, parts,
      wq1a, wq1b, r1(pq["b1"]), pq["W2"], r1(pq["b2"]),
      r1(pq["gamma"]), r1(pq["beta"]),
      po["W1"], r1(po["b1"]), po["W2"], r1(po["b2"]))

    return (el, nl, nf)


# speedup vs baseline: 1.0359x; 1.0359x over previous
import functools

import jax
import jax.numpy as jnp
from jax import lax
from jax.experimental import pallas as pl
from jax.experimental.pallas import tpu as pltpu
from jax.experimental.pallas import tpu_sc as plsc

N = 10000
E = 320000
DE = 16
H = 128
HP = H // 2

NW = 32
CBB = 128
CBD = 128
NCK = E // CBD
BCT = NCK // NW
XTR = NCK - BCT * NW
EH = E // 2
NCKH = NCK // 2
BCTH = NCKH // 16
XTRH = NCKH - BCTH * 16
EBH = 3200
NPAD = 10240
RPT = NPAD // 16
ZB = 128
NB = 2000
EB = 2000


def _ln(h, g, b):
    mu = jnp.mean(h, axis=-1, keepdims=True)
    var = jnp.mean((h - mu) ** 2, axis=-1, keepdims=True)
    return (h - mu) * lax.rsqrt(var + 1e-5) * g + b


def _swish(x):
    return x * jax.nn.sigmoid(x)


def _dot(a, b):
    return jnp.dot(a, b, preferred_element_type=jnp.float32)


def _dotb(a, b):
    return jnp.dot(a.astype(jnp.bfloat16), b.astype(jnp.bfloat16),
                   preferred_element_type=jnp.float32)


def _pack_pair(p):
    ua = lax.bitcast_convert_type(p[:, :HP], jnp.uint32)
    ub = lax.bitcast_convert_type(p[:, HP:], jnp.uint32)
    lo = (ua + jnp.uint32(0x8000)) >> 16
    hi = (ub + jnp.uint32(0x8000)) & jnp.uint32(0xFFFF0000)
    return lax.bitcast_convert_type(lo | hi, jnp.float32)


def _unpack_pair(p_ref):
    u = lax.bitcast_convert_type(p_ref[...], jnp.uint32)
    lo = lax.bitcast_convert_type(u << 16, jnp.float32)
    hi = lax.bitcast_convert_type(u & jnp.uint32(0xFFFF0000), jnp.float32)
    return lo, hi


def _node_embed_body(x_ref, w1_ref, b1_ref, w2_ref, b2_ref, gm_ref, bt_ref,
                     ws_ref, wr_ref, nf_ref, ps_ref, pr_ref):
    h = _dot(x_ref[...], w1_ref[...]) + b1_ref[...]
    h = _swish(h)
    h = _dot(h, w2_ref[...]) + b2_ref[...]
    nf = _ln(h, gm_ref[...], bt_ref[...])
    nf_ref[...] = nf
    ps_ref[...] = _pack_pair(_dot(nf, ws_ref[...]))
    pr_ref[...] = _pack_pair(_dot(nf, wr_ref[...]))


def _unpack_pair(p_ref):
    u = lax.bitcast_convert_type(p_ref[...], jnp.uint32)
    lo = lax.bitcast_convert_type(u << 16, jnp.float32)
    hi = lax.bitcast_convert_type(u & jnp.uint32(0xFFFF0000), jnp.float32)
    return lo, hi


def _edge_body(xl_ref, xh_ref, gs_ref, gr_ref, we1, be1, we2, be2, ge, bte,
               wp1, bp1, wp2, bp2, gp, btp, out_ref):
    x = jnp.concatenate([xl_ref[...], xh_ref[...]], axis=0)
    h = _dot(x, we1[...]) + be1[...]
    h = _swish(h)
    h = _dotb(h, we2[...]) + be2[...]
    ef = _ln(h, ge[...], bte[...])
    slo, shi = _unpack_pair(gs_ref)
    rlo, rhi = _unpack_pair(gr_ref)
    g = jnp.concatenate(
        [jnp.concatenate([slo[:, :HP] + rlo[:, :HP],
                          shi[:, :HP] + rhi[:, :HP]], axis=1),
         jnp.concatenate([slo[:, HP:] + rlo[:, HP:],
                          shi[:, HP:] + rhi[:, HP:]], axis=1)], axis=0)
    z = _dotb(ef, wp1[...]) + g + bp1[...]
    z = _swish(z)
    o = _dotb(z, wp2[...]) + bp2[...]
    out_ref[...] = _ln(o, gp[...], btp[...]).reshape(2, EBH, H)


def _node_update_body(nf_ref, a0_ref, a1_ref, wn1a, wn1b, b1n, wn2, b2n,
                      gn, btn, wo1, bo1, wo2, bo2, out_ref):
    nf = nf_ref[...]
    agg = a0_ref[0] + a1_ref[0]
    h = _dot(nf, wn1a[...]) + _dot(agg, wn1b[...]) + b1n[...]
    h = _swish(h)
    y = _ln(_dot(h, wn2[...]) + b2n[...], gn[...], btn[...])
    r = y + nf
    o = _swish(_dot(r, wo1[...]) + bo1[...])
    out_ref[...] = _dot(o, wo2[...]) + bo2[...]


@functools.cache
def _sc_kernels():
    mesh = plsc.VectorSubcoreMesh(core_axis_name="c", subcore_axis_name="s")

    @functools.partial(
        pl.kernel,
        mesh=mesh,
        out_type=[jax.ShapeDtypeStruct((EH, H), jnp.float32),
                  jax.ShapeDtypeStruct((EH, H), jnp.float32)],
        scratch_types=[
            pltpu.VMEM((BCTH * CBB,), jnp.int32),
            pltpu.VMEM((BCTH * CBB,), jnp.int32),
            pltpu.VMEM((CBB,), jnp.int32),
            pltpu.VMEM((CBB,), jnp.int32),
            pltpu.VMEM((CBB, HP), jnp.float32),
            pltpu.VMEM((CBB, HP), jnp.float32),
            pltpu.VMEM((CBB, HP), jnp.float32),
            pltpu.VMEM((CBB, HP), jnp.float32),
        ] + [pltpu.SemaphoreType.DMA] * 8,
        compiler_params=pltpu.CompilerParams(use_tc_tiling_on_sc=False),
    )
    def _sc_gather(sidx, ridx, tabs, tabr, gs, gr,
                   sidx_v, ridx_v, stail_v, rtail_v, rs0, rs1, rr0, rr1,
                   sgs0, sgs1, sgr0, sgr1, sws0, sws1, swr0, swr1):
        c = lax.axis_index("c")
        s = lax.axis_index("s")
        wid = s * 2 + c
        half = wid // 16
        widh = wid % 16
        starth = BCTH * widh + jnp.minimum(widh, XTRH)
        ebase = half * EH + starth * CBB
        pltpu.sync_copy(sidx.at[pl.ds(ebase, BCTH * CBB)], sidx_v)
        pltpu.sync_copy(ridx.at[pl.ds(ebase, BCTH * CBB)], ridx_v)
        col = half * HP
        rows_s = (rs0, rs1)
        rows_r = (rr0, rr1)
        sem_gs = (sgs0, sgs1)
        sem_gr = (sgr0, sgr1)
        sem_ws = (sws0, sws1)
        sem_wr = (swr0, swr1)

        def issue_g(j, b):
            pltpu.async_copy(tabs.at[sidx_v.at[pl.ds(j * CBB, CBB)]],
                             rows_s[b], sem_gs[b])
            pltpu.async_copy(tabr.at[ridx_v.at[pl.ds(j * CBB, CBB)]],
                             rows_r[b], sem_gr[b])

        def wait_g(b):
            pltpu.make_async_copy(tabs.at[pl.ds(0, CBB)], rows_s[b],
                                  sem_gs[b]).wait()
            pltpu.make_async_copy(tabr.at[pl.ds(0, CBB)], rows_r[b],
                                  sem_gr[b]).wait()

        def issue_w(j, b):
            r0w = (starth + j) * CBB
            pltpu.async_copy(rows_s[b],
                             gs.at[pl.ds(r0w, CBB), pl.ds(col, HP)],
                             sem_ws[b])
            pltpu.async_copy(rows_r[b],
                             gr.at[pl.ds(r0w, CBB), pl.ds(col, HP)],
                             sem_wr[b])

        def wait_w(b):
            pltpu.make_async_copy(rows_s[b],
                                  gs.at[pl.ds(0, CBB), pl.ds(0, HP)],
                                  sem_ws[b]).wait()
            pltpu.make_async_copy(rows_r[b],
                                  gr.at[pl.ds(0, CBB), pl.ds(0, HP)],
                                  sem_wr[b]).wait()

        issue_g(0, 0)

        def body(g, carry):
            j0 = 2 * g

            @pl.when(g >= 1)
            def _():
                wait_w(1)

            issue_g(j0 + 1, 1)
            wait_g(0)
            issue_w(j0, 0)

            @pl.when(g <= BCTH // 2 - 2)
            def _():
                wait_w(0)
                issue_g(j0 + 2, 0)

            wait_g(1)
            issue_w(j0 + 1, 1)
            return carry

        lax.fori_loop(0, BCTH // 2, body, 0)
        wait_w(0)
        wait_w(1)

        @pl.when(widh < XTRH)
        def _():
            jt = starth + BCTH
            et = half * EH + jt * CBB
            pltpu.sync_copy(sidx.at[pl.ds(et, CBB)], stail_v)
            pltpu.sync_copy(ridx.at[pl.ds(et, CBB)], rtail_v)
            pltpu.async_copy(tabs.at[stail_v], rs0, sgs0).wait()
            pltpu.async_copy(tabr.at[rtail_v], rr0, sgr0).wait()
            pltpu.sync_copy(rs0, gs.at[pl.ds(jt * CBB, CBB), pl.ds(col, HP)])
            pltpu.sync_copy(rr0, gr.at[pl.ds(jt * CBB, CBB), pl.ds(col, HP)])

    @functools.partial(
        pl.kernel,
        mesh=mesh,
        out_type=jax.ShapeDtypeStruct((2, NPAD, H), jnp.float32),
        scratch_types=[
            pltpu.VMEM((BCT + 2, CBD), jnp.int32),
            pltpu.VMEM((ZB, H), jnp.float32),
            pltpu.VMEM((ZB, H), jnp.float32),
            pltpu.VMEM_SHARED((NPAD, H), jnp.float32),
            pltpu.SemaphoreType.DMA,
            pltpu.SemaphoreType.DMA,
        ],
    )
    def _sc_segsum(ridx, el, zer, out, ridx_v, r0, r1, agg_sp,
                   sem0, sem1):
        c = lax.axis_index("c")
        s = lax.axis_index("s")
        wid = s * 2 + c
        start = BCT * wid + jnp.minimum(wid, XTR)
        pltpu.sync_copy(ridx.at[wid], ridx_v)
        pltpu.sync_copy(zer, r0)
        for z in range(RPT // ZB):
            pltpu.sync_copy(r0, agg_sp.at[pl.ds(s * RPT + z * ZB, ZB)])
        plsc.subcore_barrier()
        rows = (r0, r1)
        sems = (sem0, sem1)

        def issue_r(j, b):
            pltpu.async_copy(el.at[pl.ds((start + j) * CBD, CBD)],
                             rows[b], sems[b])

        def wait_r(b):
            pltpu.make_async_copy(el.at[pl.ds(0, CBD)], rows[b],
                                  sems[b]).wait()

        issue_r(0, 0)

        def body(g, carry):
            j0 = 2 * g
            issue_r(j0 + 1, 1)
            wait_r(0)
            pltpu.sync_copy(rows[0], agg_sp.at[ridx_v.at[j0]], add=True)

            @pl.when(g <= BCT // 2 - 2)
            def _():
                issue_r(j0 + 2, 0)

            wait_r(1)
            pltpu.sync_copy(rows[1], agg_sp.at[ridx_v.at[j0 + 1]], add=True)
            return carry

        lax.fori_loop(0, BCT // 2, body, 0)

        @pl.when(wid < XTR)
        def _():
            pltpu.sync_copy(el.at[pl.ds((start + BCT) * CBD, CBD)], r0)
            pltpu.sync_copy(r0, agg_sp.at[ridx_v.at[BCT]], add=True)

        plsc.subcore_barrier()
        for z in range(RPT // ZB):
            pltpu.sync_copy(agg_sp.at[pl.ds(s * RPT + z * ZB, ZB)], r0)
            pltpu.sync_copy(r0, out.at[c, pl.ds(s * RPT + z * ZB, ZB)])

    return _sc_gather, _sc_segsum


def _row_spec(block, idx_fn):
    return pl.BlockSpec(block, idx_fn)


def kernel(edge_idx, edge_features, node_features, params):
    pe = params["embed_edge"]
    pn = params["embed_node"]
    pp = params["proc_edge"]
    pq = params["proc_node"]
    po = params["node_out"]

    r1 = lambda v: v.reshape(1, H)
    senders = edge_idx[0]
    receivers = edge_idx[1]

    w1e = pp["W1"][:H]
    w1s = pp["W1"][H:2 * H]
    w1r = pp["W1"][2 * H:]
    wq1a = pq["W1"][:H]
    wq1b = pq["W1"][H:]

    wspec = lambda shape: pl.BlockSpec(shape, lambda i: (0, 0))

    nf, tabs, tabr = pl.pallas_call(
        _node_embed_body,
        grid=(N // NB,),
        in_specs=[
            _row_spec((NB, H), lambda i: (i, 0)),
            wspec((H, H)), wspec((1, H)), wspec((H, H)), wspec((1, H)),
            wspec((1, H)), wspec((1, H)), wspec((H, H)), wspec((H, H)),
        ],
        out_specs=[_row_spec((NB, H), lambda i: (i, 0)),
                   _row_spec((NB, HP), lambda i: (i, 0)),
                   _row_spec((NB, HP), lambda i: (i, 0))],
        out_shape=[jax.ShapeDtypeStruct((N, H), jnp.float32),
                   jax.ShapeDtypeStruct((N, HP), jnp.float32),
                   jax.ShapeDtypeStruct((N, HP), jnp.float32)],
    )(node_features, pn["W1"], r1(pn["b1"]), pn["W2"], r1(pn["b2"]),
      r1(pn["gamma"]), r1(pn["beta"]), w1s, w1r)

    sc_gather, sc_segsum = _sc_kernels()
    gs, gr = sc_gather(senders, receivers, tabs, tabr)

    el3 = pl.pallas_call(
        _edge_body,
        grid=(EH // EBH,),
        in_specs=[
            pl.BlockSpec((EBH, DE), lambda i: (i, 0)),
            pl.BlockSpec((EBH, DE), lambda i: (EH // EBH + i, 0)),
            pl.BlockSpec((EBH, H), lambda i: (i, 0)),
            pl.BlockSpec((EBH, H), lambda i: (i, 0)),
            wspec((DE, H)), wspec((1, H)), wspec((H, H)), wspec((1, H)),
            wspec((1, H)), wspec((1, H)),
            wspec((H, H)), wspec((1, H)), wspec((H, H)), wspec((1, H)),
            wspec((1, H)), wspec((1, H)),
        ],
        out_specs=pl.BlockSpec((2, EBH, H), lambda i: (0, i, 0)),
        out_shape=jax.ShapeDtypeStruct((2, EH, H), jnp.float32),
    )(edge_features, edge_features, gs, gr,
      pe["W1"], r1(pe["b1"]), pe["W2"], r1(pe["b2"]),
      r1(pe["gamma"]), r1(pe["beta"]),
      w1e, r1(pp["b1"]), pp["W2"], r1(pp["b2"]),
      r1(pp["gamma"]), r1(pp["beta"]))
    el = el3.reshape(E, H)

    starts = BCT * jnp.arange(NW) + jnp.minimum(jnp.arange(NW), XTR)
    ridx_d = jnp.pad(receivers.reshape(NCK, CBD), ((0, 8), (0, 0)))[
        starts[:, None] + jnp.arange(BCT + 2)[None, :]]
    zer = jnp.zeros((ZB, H), jnp.float32)
    parts = sc_segsum(ridx_d, el, zer)

    nl = pl.pallas_call(
        _node_update_body,
        grid=(N // NB,),
        in_specs=[
            _row_spec((NB, H), lambda i: (i, 0)),
            pl.BlockSpec((1, NB, H), lambda i: (0, i, 0)),
            pl.BlockSpec((1, NB, H), lambda i: (1, i, 0)),
            wspec((H, H)), wspec((H, H)), wspec((1, H)),
            wspec((H, H)), wspec((1, H)), wspec((1, H)), wspec((1, H)),
            wspec((H, H)), wspec((1, H)), wspec((H, H)), wspec((1, H)),
        ],
        out_specs=_row_spec((NB, H), lambda i: (i, 0)),
        out_shape=jax.ShapeDtypeStruct((N, H), jnp.float32),
    )(nf, parts, parts,
      wq1a, wq1b, r1(pq["b1"]), pq["W2"], r1(pq["b2"]),
      r1(pq["gamma"]), r1(pq["beta"]),
      po["W1"], r1(po["b1"]), po["W2"], r1(po["b2"]))

    return (el, nl, nf)
